# baseline (speedup 1.0000x reference)
import functools

import jax
import jax.numpy as jnp
from jax import lax
from jax.experimental import pallas as pl
from jax.experimental.pallas import tpu as pltpu
from jax.experimental.pallas import tpu_sc as plsc

_B, _C, _T = 16, 512, 2048
_HALF = _C // 2
_GROUP = 8
_NBUF = 4
_WINDOW = 8

_MASKED = [
    31, 35, 45, 63, 85, 99, 112, 114, 117, 121, 130, 139, 144, 148, 152,
    174, 176, 179, 188, 189, 197, 257, 263, 268, 272, 304, 309, 312, 315,
    318, 325, 356, 366, 398, 409, 410, 417, 429, 441, 446, 448, 462, 480,
    481, 487, 493, 495, 499, 501, 507, 509,
]


def _half_plan(h):
    lo = h * _HALF
    mset = set(c - lo for c in _MASKED if lo <= c < lo + _HALF)
    clean_runs, dirty = [], []
    run_start = None
    for g in range(_HALF // _GROUP):
        rows = range(g * _GROUP, (g + 1) * _GROUP)
        mk = tuple(r - g * _GROUP for r in rows if r in mset)
        if mk:
            if run_start is not None:
                clean_runs.append((run_start, g * _GROUP - run_start))
                run_start = None
            dirty.append((g * _GROUP, mk))
        elif run_start is None:
            run_start = g * _GROUP
    if run_start is not None:
        clean_runs.append((run_start, _HALF - run_start))
    return clean_runs, dirty


_PLANS = (_half_plan(0), _half_plan(1))

_mesh = plsc.VectorSubcoreMesh(core_axis_name="c", subcore_axis_name="s")

_Z16 = None


@functools.partial(
    pl.kernel,
    mesh=_mesh,
    out_type=jax.ShapeDtypeStruct((_B * _C, _T), jnp.float32),
    scratch_types=(
        [pltpu.VMEM((_GROUP, _T), jnp.float32) for _ in range(_NBUF)]
        + [pltpu.SemaphoreType.DMA for _ in range(2 * _NBUF)]
        + [pltpu.SemaphoreType.DMA]
    ),
)
def _sc_mask_copy(x_hbm, out_hbm,
                  vb0, vb1, vb2, vb3,
                  si0, si1, si2, si3, so0, so1, so2, so3,
                  sem_c):
    b = lax.axis_index("s")
    h = lax.axis_index("c")
    bufs = (vb0, vb1, vb2, vb3)
    isems = (si0, si1, si2, si3)
    osems = (so0, so1, so2, so3)

    for hh in (0, 1):
        clean_runs, dirty = _PLANS[hh]

        @pl.when(h == hh)
        def _():
            base = b * _C + hh * _HALF
            pend = []
            for off, n in clean_runs:
                if len(pend) >= _WINDOW:
                    pend.pop(0).wait()
                pend.append(
                    pltpu.async_copy(
                        x_hbm.at[pl.ds(base + off, n)],
                        out_hbm.at[pl.ds(base + off, n)],
                        sem_c,
                    )
                )

            n_d = len(dirty)

            def start_in(i):
                off, _mk = dirty[i]
                return pltpu.async_copy(
                    x_hbm.at[pl.ds(base + off, _GROUP)],
                    bufs[i % _NBUF],
                    isems[i % _NBUF],
                )

            in_h = {}
            out_h = {}
            out_waited = set()
            for j in range(min(_NBUF - 1, n_d)):
                in_h[j] = start_in(j)
            for i in range(n_d):
                nxt = _NBUF - 1 + i
                if nxt < n_d:
                    if i > 0:
                        out_h[i - 1].wait()
                        out_waited.add(i - 1)
                    in_h[nxt] = start_in(nxt)
                in_h[i].wait()
                off, mk = dirty[i]
                buf = bufs[i % _NBUF]
                for j in mk:
                    z = jnp.zeros((16,), jnp.float32)

                    def zero_row(k, carry, buf=buf, j=j, z=z):
                        buf[j, pl.ds(k * 64, 16)] = z
                        buf[j, pl.ds(k * 64 + 16, 16)] = z
                        buf[j, pl.ds(k * 64 + 32, 16)] = z
                        buf[j, pl.ds(k * 64 + 48, 16)] = z
                        return carry

                    lax.fori_loop(0, _T // 64, zero_row, 0)
                out_h[i] = pltpu.async_copy(
                    buf,
                    out_hbm.at[pl.ds(base + off, _GROUP)],
                    osems[i % _NBUF],
                )
            for i in range(n_d):
                if i not in out_waited:
                    out_h[i].wait()
            for cp in pend:
                cp.wait()


def kernel(x):
    B, C, T = x.shape
    x2 = x.reshape(B * C, T)
    out = _sc_mask_copy(x2)
    return out.reshape(B, C, T)


# speedup vs baseline: 14.1493x; 14.1493x over previous
import functools

import jax
import jax.numpy as jnp
from jax import lax
from jax.experimental import pallas as pl
from jax.experimental.pallas import tpu as pltpu
from jax.experimental.pallas import tpu_sc as plsc

_B, _C, _T = 16, 512, 2048
_HALF = _C // 2
_CHUNK = 16
_NBUF = 3

_MASKED = [
    31, 35, 45, 63, 85, 99, 112, 114, 117, 121, 130, 139, 144, 148, 152,
    174, 176, 179, 188, 189, 197, 257, 263, 268, 272, 304, 309, 312, 315,
    318, 325, 356, 366, 398, 409, 410, 417, 429, 441, 446, 448, 462, 480,
    481, 487, 493, 495, 499, 501, 507, 509,
]


def _half_plan(h):
    lo = h * _HALF
    mset = set(c - lo for c in _MASKED if lo <= c < lo + _HALF)
    chunks = []
    for g in range(_HALF // _CHUNK):
        rows = range(g * _CHUNK, (g + 1) * _CHUNK)
        chunks.append(tuple(r - g * _CHUNK for r in rows if r in mset))
    return chunks


_PLANS = (_half_plan(0), _half_plan(1))

_mesh = plsc.VectorSubcoreMesh(core_axis_name="c", subcore_axis_name="s")


@functools.partial(
    pl.kernel,
    mesh=_mesh,
    out_type=jax.ShapeDtypeStruct((_B * _C, _T), jnp.float32),
    scratch_types=(
        [pltpu.VMEM((_CHUNK, _T), jnp.float32) for _ in range(_NBUF)]
        + [pltpu.SemaphoreType.DMA for _ in range(2 * _NBUF)]
    ),
)
def _sc_mask_copy(x_hbm, out_hbm,
                  vb0, vb1, vb2,
                  si0, si1, si2, so0, so1, so2):
    b = lax.axis_index("s")
    h = lax.axis_index("c")
    bufs = (vb0, vb1, vb2)
    isems = (si0, si1, si2)
    osems = (so0, so1, so2)

    for hh in (0, 1):
        chunks = _PLANS[hh]
        n_ck = len(chunks)

        @pl.when(h == hh)
        def _():
            base = b * _C + hh * _HALF

            def start_in(i):
                return pltpu.async_copy(
                    x_hbm.at[pl.ds(base + i * _CHUNK, _CHUNK)],
                    bufs[i % _NBUF],
                    isems[i % _NBUF],
                )

            in_h = {}
            out_h = {}
            out_waited = set()
            for j in range(min(_NBUF - 1, n_ck)):
                in_h[j] = start_in(j)
            for i in range(n_ck):
                nxt = _NBUF - 1 + i
                if nxt < n_ck:
                    if i > 0:
                        out_h[i - 1].wait()
                        out_waited.add(i - 1)
                    in_h[nxt] = start_in(nxt)
                in_h[i].wait()
                buf = bufs[i % _NBUF]
                for j in chunks[i]:
                    z = jnp.zeros((16,), jnp.float32)

                    def zero_row(k, carry, buf=buf, j=j, z=z):
                        buf[j, pl.ds(k * 64, 16)] = z
                        buf[j, pl.ds(k * 64 + 16, 16)] = z
                        buf[j, pl.ds(k * 64 + 32, 16)] = z
                        buf[j, pl.ds(k * 64 + 48, 16)] = z
                        return carry

                    lax.fori_loop(0, _T // 64, zero_row, 0)
                out_h[i] = pltpu.async_copy(
                    buf,
                    out_hbm.at[pl.ds(base + i * _CHUNK, _CHUNK)],
                    osems[i % _NBUF],
                )
            for i in range(n_ck):
                if i not in out_waited:
                    out_h[i].wait()


def kernel(x):
    B, C, T = x.shape
    x2 = x.reshape(B * C, T)
    out = _sc_mask_copy(x2)
    return out.reshape(B, C, T)
